# Initial kernel scaffold; baseline (speedup 1.0000x reference)
import jax
import jax.numpy as jnp
from jax.experimental import pallas as pl


def kernel(user_index, candidate_news_index, label, user_emb, item_emb, edge_src, edge_dst):
    raise NotImplementedError("write your pallas kernel here")


# speedup vs baseline: 6.2423x; 6.2423x over previous
import functools

import jax
import jax.numpy as jnp
from jax import lax
from jax.experimental import pallas as pl
from jax.experimental.pallas import tpu as pltpu
from jax.experimental.pallas import tpu_sc as plsc

N_USERS = 5000
N_ITEMS = 5000
DIM = 128
N_LAYERS = 3
N_EDGES = 320000
L2_COEF = 1e-4
BATCH = 1024
K_CAND = 5

NC, NS = 2, 16
NW = NC * NS
HALF = 5120
N_PAD = 2 * HALF
ZROWS = HALF // NS
EPT = N_EDGES // NW
CH = 128
NFULL = EPT // CH
REM = EPT - NFULL * CH

@functools.cache
def _make_layer():
  mesh = plsc.VectorSubcoreMesh(
      core_axis_name="c", subcore_axis_name="s",
      num_cores=NC, num_subcores=NS)

  @functools.partial(
      pl.kernel,
      out_type=jax.ShapeDtypeStruct((N_PAD, DIM), jnp.float32),
      mesh=mesh,
      scratch_types=[
          pltpu.VMEM_SHARED((N_PAD, DIM), jnp.float32),
          pltpu.VMEM((CH,), jnp.int32),
          pltpu.VMEM((CH,), jnp.int32),
          pltpu.VMEM((CH, DIM), jnp.float32),
          pltpu.VMEM((REM,), jnp.int32),
          pltpu.VMEM((REM,), jnp.int32),
          pltpu.VMEM((REM, DIM), jnp.float32),
          pltpu.SemaphoreType.DMA,
      ],
  )
  def _layer(t_in, esrc, edst, zin, t_out,
             acc, sidx, didx, rows, sidx_r, didx_r, rows_r, sem):
    c = lax.axis_index("c")
    s = lax.axis_index("s")
    half_base = (1 - c) * HALF
    zb = half_base + s * ZROWS
    pltpu.sync_copy(zin, acc.at[pl.ds(zb, ZROWS)])
    plsc.subcore_barrier()

    ebase = (c * NS + s) * EPT

    def chunk(i, _):
      b = ebase + i * CH
      pltpu.sync_copy(esrc.at[pl.ds(b, CH)], sidx)
      pltpu.sync_copy(edst.at[pl.ds(b, CH)], didx)
      pltpu.async_copy(t_in.at[sidx], rows, sem).wait()
      pltpu.sync_copy(rows, acc.at[didx], add=True)
      return 0

    lax.fori_loop(0, NFULL, chunk, 0)

    b = ebase + NFULL * CH
    pltpu.sync_copy(esrc.at[pl.ds(b, REM)], sidx_r)
    pltpu.sync_copy(edst.at[pl.ds(b, REM)], didx_r)
    pltpu.async_copy(t_in.at[sidx_r], rows_r, sem).wait()
    pltpu.sync_copy(rows_r, acc.at[didx_r], add=True)

    plsc.subcore_barrier()
    pltpu.sync_copy(acc.at[pl.ds(zb, ZROWS)], t_out.at[pl.ds(zb, ZROWS)])

  return _layer


N_GATHER = BATCH * (1 + K_CAND)
GPT = N_GATHER // NW
GCH = GPT // 2


@functools.cache
def _make_gather4():
  mesh = plsc.VectorSubcoreMesh(
      core_axis_name="c", subcore_axis_name="s",
      num_cores=NC, num_subcores=NS)

  @functools.partial(
      pl.kernel,
      out_type=[jax.ShapeDtypeStruct((N_GATHER, DIM), jnp.float32)] * 4,
      mesh=mesh,
      scratch_types=[
          pltpu.VMEM((GCH,), jnp.int32),
          pltpu.VMEM((GCH, DIM), jnp.float32),
          pltpu.SemaphoreType.DMA,
      ],
  )
  def _gather4(t0, t1, t2, t3, gidx, g0, g1, g2, g3, idx, rows, sem):
    wid = lax.axis_index("c") * NS + lax.axis_index("s")
    for q in range(2):
      base = wid * GPT + q * GCH
      pltpu.sync_copy(gidx.at[pl.ds(base, GCH)], idx)
      for t_hbm, g_hbm in ((t0, g0), (t1, g1), (t2, g2), (t3, g3)):
        pltpu.async_copy(t_hbm.at[idx], rows, sem).wait()
        pltpu.sync_copy(rows, g_hbm.at[pl.ds(base, GCH)])

  return _gather4


def _finalize(g0, g1, g2, g3, label, tot_ref, scores_ref, rec_ref, emb_ref):
    u = 0.25 * (g0[0:BATCH, :] + g1[0:BATCH, :]
                + g2[0:BATCH, :] + g3[0:BATCH, :])
    reg = jnp.sum(u * u)
    cols = []
    for k in range(K_CAND):
        o = BATCH + k * BATCH
        ik = 0.25 * (g0[o:o + BATCH, :] + g1[o:o + BATCH, :]
                     + g2[o:o + BATCH, :] + g3[o:o + BATCH, :])
        reg = reg + jnp.sum(ik * ik)
        cols.append(jnp.sum(u * ik, axis=1, keepdims=True))
    scores = jnp.concatenate(cols, axis=1)

    m = jnp.max(scores, axis=1, keepdims=True)
    e = jnp.exp(scores - m)
    probs = e / jnp.sum(e, axis=1, keepdims=True)

    lbl = label[...]
    iota_k = lax.broadcasted_iota(jnp.int32, (BATCH, K_CAND), 1)
    lmax = jnp.max(lbl, axis=1, keepdims=True)
    tgt = jnp.min(jnp.where(lbl == lmax, iota_k, K_CAND),
                  axis=1, keepdims=True)

    m2 = jnp.max(probs, axis=1, keepdims=True)
    logp = (probs - m2
            - jnp.log(jnp.sum(jnp.exp(probs - m2), axis=1, keepdims=True)))
    chosen = jnp.sum(jnp.where(iota_k == tgt, logp, 0.0), axis=1)
    rec = -jnp.sum(chosen) / BATCH
    emb = L2_COEF * reg * 0.5 / BATCH

    scores_ref[...] = scores
    tot_ref[...] = jnp.reshape(rec + emb, (1, 1))
    rec_ref[...] = jnp.reshape(rec, (1, 1))
    emb_ref[...] = jnp.reshape(emb, (1, 1))


_finalize_call = pl.pallas_call(
    _finalize,
    out_shape=[
        jax.ShapeDtypeStruct((1, 1), jnp.float32),
        jax.ShapeDtypeStruct((BATCH, K_CAND), jnp.float32),
        jax.ShapeDtypeStruct((1, 1), jnp.float32),
        jax.ShapeDtypeStruct((1, 1), jnp.float32),
    ],
)


def kernel(user_index, candidate_news_index, label,
           user_emb, item_emb, edge_src, edge_dst):
    t0 = jnp.zeros((N_PAD, DIM), jnp.float32)
    t0 = lax.dynamic_update_slice(t0, user_emb, (0, 0))
    t0 = lax.dynamic_update_slice(t0, item_emb, (HALF, 0))

    esrc = edge_src.astype(jnp.int32)
    edst = edge_dst.astype(jnp.int32)
    esrc = esrc + jnp.where(esrc >= N_USERS, HALF - N_USERS, 0)
    edst = edst + jnp.where(edst >= N_USERS, HALF - N_USERS, 0)
    zin = jnp.zeros((ZROWS, DIM), jnp.float32)

    layer_fn = _make_layer()
    t1 = layer_fn(t0, esrc, edst, zin)
    t2 = layer_fn(t1, esrc, edst, zin)
    t3 = layer_fn(t2, esrc, edst, zin)

    gidx = jnp.concatenate(
        [user_index.astype(jnp.int32)]
        + [HALF + candidate_news_index[:, k].astype(jnp.int32)
           for k in range(K_CAND)])
    g0, g1, g2, g3 = _make_gather4()(t0, t1, t2, t3, gidx)

    tot, scores, rec, emb = _finalize_call(g0, g1, g2, g3, label)
    return (tot[0, 0], scores, rec[0, 0], emb[0, 0])


# speedup vs baseline: 13.5949x; 2.1779x over previous
import functools

import jax
import jax.numpy as jnp
from jax import lax
from jax.experimental import pallas as pl
from jax.experimental.pallas import tpu as pltpu
from jax.experimental.pallas import tpu_sc as plsc

N_USERS = 5000
N_ITEMS = 5000
DIM = 128
N_LAYERS = 3
N_EDGES = 320000
L2_COEF = 1e-4
BATCH = 1024
K_CAND = 5

NC, NS = 2, 16
NW = NC * NS
HALF = 5120
N_PAD = 2 * HALF
PADROWS = 128
N_ACC = N_PAD + PADROWS
ZROWS = HALF // NS
EPT = N_EDGES // NW
CH = 128
NCH = (EPT + CH - 1) // CH
EPT_P = NCH * CH
PADE = EPT_P - EPT


@functools.cache
def _make_layer():
  mesh = plsc.VectorSubcoreMesh(
      core_axis_name="c", subcore_axis_name="s",
      num_cores=NC, num_subcores=NS)

  @functools.partial(
      pl.kernel,
      out_type=jax.ShapeDtypeStruct((N_PAD, DIM), jnp.float32),
      mesh=mesh,
      scratch_types=[
          pltpu.VMEM_SHARED((N_ACC, DIM), jnp.float32),
          pltpu.VMEM(((NCH + 1) * CH,), jnp.int32),
          pltpu.VMEM((CH,), jnp.int32),
          pltpu.VMEM((CH,), jnp.int32),
          pltpu.VMEM((CH,), jnp.int32),
          pltpu.VMEM((CH,), jnp.int32),
          pltpu.VMEM((CH, DIM), jnp.float32),
          pltpu.VMEM((CH, DIM), jnp.float32),
          pltpu.SemaphoreType.DMA,
          pltpu.SemaphoreType.DMA,
          pltpu.SemaphoreType.DMA,
      ],
  )
  def _layer(t_in, epk, zin, t_out,
             acc, pk, sidx0, didx0, sidx1, didx1, rows0, rows1,
             sem0, sem1, semz):
    c = lax.axis_index("c")
    s = lax.axis_index("s")
    w = c * NS + s
    zb = (1 - c) * HALF + s * ZROWS

    pltpu.sync_copy(epk.at[w], pk)
    zero_dma = pltpu.async_copy(zin, acc.at[pl.ds(zb, ZROWS)], semz)

    def unpack(j, sidx, didx):
      def step(i, _):
        v = pk[pl.ds(j * CH + i * 16, 16)]
        sidx[pl.ds(i * 16, 16)] = v & 0xFFFF
        didx[pl.ds(i * 16, 16)] = lax.shift_right_logical(v, 16)
        return 0
      lax.fori_loop(0, CH // 16, step, 0)

    unpack(0, sidx0, didx0)
    pltpu.async_copy(t_in.at[sidx0], rows0, sem0)
    unpack(1, sidx1, didx1)
    zero_dma.wait()
    plsc.subcore_barrier()

    def pair(k, _):
      j0 = 2 * k
      pltpu.async_copy(t_in.at[sidx1], rows1, sem1)
      pltpu.make_async_copy(t_in.at[sidx0], rows0, sem0).wait()
      pltpu.sync_copy(rows0, acc.at[didx0], add=True)
      unpack(j0 + 2, sidx0, didx0)
      pltpu.async_copy(t_in.at[sidx0], rows0, sem0)
      pltpu.make_async_copy(t_in.at[sidx1], rows1, sem1).wait()
      pltpu.sync_copy(rows1, acc.at[didx1], add=True)
      unpack(j0 + 3, sidx1, didx1)
      return 0

    lax.fori_loop(0, (NCH - 1) // 2, pair, 0)

    pltpu.make_async_copy(t_in.at[sidx0], rows0, sem0).wait()
    pltpu.sync_copy(rows0, acc.at[didx0], add=True)

    plsc.subcore_barrier()
    pltpu.sync_copy(acc.at[pl.ds(zb, ZROWS)], t_out.at[pl.ds(zb, ZROWS)])

  return _layer


N_GATHER = BATCH * (1 + K_CAND)
GPT = N_GATHER // NW
GCH = GPT // 2


@functools.cache
def _make_gather4():
  mesh = plsc.VectorSubcoreMesh(
      core_axis_name="c", subcore_axis_name="s",
      num_cores=NC, num_subcores=NS)

  @functools.partial(
      pl.kernel,
      out_type=[jax.ShapeDtypeStruct((N_GATHER, DIM), jnp.float32)] * 4,
      mesh=mesh,
      scratch_types=[
          pltpu.VMEM((GCH,), jnp.int32),
          pltpu.VMEM((GCH,), jnp.int32),
          pltpu.VMEM((GCH, DIM), jnp.float32),
          pltpu.VMEM((GCH, DIM), jnp.float32),
          pltpu.SemaphoreType.DMA,
          pltpu.SemaphoreType.DMA,
      ],
  )
  def _gather4(t0, t1, t2, t3, gidx, g0, g1, g2, g3,
               idx_a, idx_b, rows_a, rows_b, sem_a, sem_b):
    wid = lax.axis_index("c") * NS + lax.axis_index("s")
    base_a = wid * GPT
    base_b = base_a + GCH
    pltpu.sync_copy(gidx.at[pl.ds(base_a, GCH)], idx_a)
    pltpu.sync_copy(gidx.at[pl.ds(base_b, GCH)], idx_b)
    tables = (t0, t1, t2, t3)
    outs = (g0, g1, g2, g3)
    steps = [(tables[j % 4], outs[j % 4],
              idx_a if j < 4 else idx_b,
              base_a if j < 4 else base_b) for j in range(8)]
    bufs = ((rows_a, sem_a), (rows_b, sem_b))
    t_hbm, _, idx, _ = steps[0]
    pltpu.async_copy(t_hbm.at[idx], rows_a, sem_a)
    for j in range(8):
      rows, sem = bufs[j % 2]
      if j < 7:
        t_n, _, idx_n, _ = steps[j + 1]
        rows_n, sem_n = bufs[(j + 1) % 2]
        pltpu.async_copy(t_n.at[idx_n], rows_n, sem_n)
      t_hbm, g_hbm, idx, base = steps[j]
      pltpu.make_async_copy(t_hbm.at[idx], rows, sem).wait()
      pltpu.sync_copy(rows, g_hbm.at[pl.ds(base, GCH)])

  return _gather4


def _finalize(g0, g1, g2, g3, label, tot_ref, scores_ref, rec_ref, emb_ref):
    u = 0.25 * (g0[0:BATCH, :] + g1[0:BATCH, :]
                + g2[0:BATCH, :] + g3[0:BATCH, :])
    reg = jnp.sum(u * u)
    cols = []
    for k in range(K_CAND):
        o = BATCH + k * BATCH
        ik = 0.25 * (g0[o:o + BATCH, :] + g1[o:o + BATCH, :]
                     + g2[o:o + BATCH, :] + g3[o:o + BATCH, :])
        reg = reg + jnp.sum(ik * ik)
        cols.append(jnp.sum(u * ik, axis=1, keepdims=True))
    scores = jnp.concatenate(cols, axis=1)

    m = jnp.max(scores, axis=1, keepdims=True)
    e = jnp.exp(scores - m)
    probs = e / jnp.sum(e, axis=1, keepdims=True)

    lbl = label[...]
    iota_k = lax.broadcasted_iota(jnp.int32, (BATCH, K_CAND), 1)
    lmax = jnp.max(lbl, axis=1, keepdims=True)
    tgt = jnp.min(jnp.where(lbl == lmax, iota_k, K_CAND),
                  axis=1, keepdims=True)

    m2 = jnp.max(probs, axis=1, keepdims=True)
    logp = (probs - m2
            - jnp.log(jnp.sum(jnp.exp(probs - m2), axis=1, keepdims=True)))
    chosen = jnp.sum(jnp.where(iota_k == tgt, logp, 0.0), axis=1)
    rec = -jnp.sum(chosen) / BATCH
    emb = L2_COEF * reg * 0.5 / BATCH

    scores_ref[...] = scores
    tot_ref[...] = jnp.reshape(rec + emb, (1, 1))
    rec_ref[...] = jnp.reshape(rec, (1, 1))
    emb_ref[...] = jnp.reshape(emb, (1, 1))


_finalize_call = pl.pallas_call(
    _finalize,
    out_shape=[
        jax.ShapeDtypeStruct((1, 1), jnp.float32),
        jax.ShapeDtypeStruct((BATCH, K_CAND), jnp.float32),
        jax.ShapeDtypeStruct((1, 1), jnp.float32),
        jax.ShapeDtypeStruct((1, 1), jnp.float32),
    ],
)


def kernel(user_index, candidate_news_index, label,
           user_emb, item_emb, edge_src, edge_dst):
    t0 = jnp.zeros((N_PAD, DIM), jnp.float32)
    t0 = lax.dynamic_update_slice(t0, user_emb, (0, 0))
    t0 = lax.dynamic_update_slice(t0, item_emb, (HALF, 0))

    esrc = edge_src.astype(jnp.int32)
    edst = edge_dst.astype(jnp.int32)
    esrc = esrc + jnp.where(esrc >= N_USERS, HALF - N_USERS, 0)
    edst = edst + jnp.where(edst >= N_USERS, HALF - N_USERS, 0)

    pad_src = (jnp.arange(NW * PADE, dtype=jnp.int32) % N_PAD).reshape(
        NW, PADE)
    pad_dst = (N_PAD + jnp.arange(NW * PADE, dtype=jnp.int32) % PADROWS
               ).reshape(NW, PADE)
    src_p = jnp.concatenate([esrc.reshape(NW, EPT), pad_src], axis=1)
    dst_p = jnp.concatenate([edst.reshape(NW, EPT), pad_dst], axis=1)
    packed = src_p | (dst_p << 16)
    packed = jnp.concatenate(
        [packed, jnp.zeros((NW, CH), jnp.int32)], axis=1)
    zin = jnp.zeros((ZROWS, DIM), jnp.float32)

    layer_fn = _make_layer()
    t1 = layer_fn(t0, packed, zin)
    t2 = layer_fn(t1, packed, zin)
    t3 = layer_fn(t2, packed, zin)

    gidx = jnp.concatenate(
        [user_index.astype(jnp.int32)]
        + [HALF + candidate_news_index[:, k].astype(jnp.int32)
           for k in range(K_CAND)])
    g0, g1, g2, g3 = _make_gather4()(t0, t1, t2, t3, gidx)

    tot, scores, rec, emb = _finalize_call(g0, g1, g2, g3, label)
    return (tot[0, 0], scores, rec[0, 0], emb[0, 0])


# speedup vs baseline: 14.9444x; 1.0993x over previous
import functools

import jax
import jax.numpy as jnp
from jax import lax
from jax.experimental import pallas as pl
from jax.experimental.pallas import tpu as pltpu
from jax.experimental.pallas import tpu_sc as plsc

N_USERS = 5000
N_ITEMS = 5000
DIM = 128
N_LAYERS = 3
N_EDGES = 320000
L2_COEF = 1e-4
BATCH = 1024
K_CAND = 5

NC, NS = 2, 16
NW = NC * NS
HALF = 5120
N_PAD = 2 * HALF
PADROWS = 128
N_ACC = HALF + PADROWS
ZROWS = HALF // NS
EPT = N_EDGES // NW
CH = 128
NBUF = 4
NCH = 80
EPT_P = NCH * CH
PADE = EPT_P - EPT


@functools.cache
def _make_layer():
  mesh = plsc.VectorSubcoreMesh(
      core_axis_name="c", subcore_axis_name="s",
      num_cores=NC, num_subcores=NS)

  @functools.partial(
      pl.kernel,
      out_type=jax.ShapeDtypeStruct((N_PAD, DIM), jnp.float32),
      mesh=mesh,
      scratch_types=[
          pltpu.VMEM_SHARED((N_ACC, DIM), jnp.float32),
          pltpu.VMEM((NCH * CH,), jnp.int32),
          [pltpu.VMEM((CH,), jnp.int32) for _ in range(NBUF)],
          [pltpu.VMEM((CH,), jnp.int32) for _ in range(NBUF)],
          [pltpu.VMEM((CH, DIM), jnp.float32) for _ in range(NBUF)],
          [pltpu.SemaphoreType.DMA for _ in range(NBUF)],
          pltpu.SemaphoreType.DMA,
      ],
  )
  def _layer(t_in, epk, zin, t_out,
             acc, pk, sidx, didx, rows, sems, semz):
    c = lax.axis_index("c")
    s = lax.axis_index("s")
    w = c * NS + s
    hb = (1 - c) * HALF + s * ZROWS
    zb = s * ZROWS

    pltpu.sync_copy(epk.at[w], pk)
    zero_dma = pltpu.async_copy(zin, acc.at[pl.ds(zb, ZROWS)], semz)

    def unpack(j, b):
      def step(i, _):
        v = pk[pl.ds(j * CH + i * 16, 16)]
        sidx[b][pl.ds(i * 16, 16)] = v & 0xFFFF
        didx[b][pl.ds(i * 16, 16)] = lax.shift_right_logical(v, 16)
        return 0
      lax.fori_loop(0, CH // 16, step, 0)

    for b in range(NBUF):
      unpack(b, b)
      pltpu.async_copy(t_in.at[sidx[b]], rows[b], sems[b])
    zero_dma.wait()
    plsc.subcore_barrier()

    def quad(k, _):
      for b in range(NBUF):
        j = NBUF * k + b
        pltpu.make_async_copy(t_in.at[sidx[b]], rows[b], sems[b]).wait()
        pltpu.sync_copy(rows[b], acc.at[didx[b]], add=True)
        unpack(j + NBUF, b)
        pltpu.async_copy(t_in.at[sidx[b]], rows[b], sems[b])
      return 0

    lax.fori_loop(0, NCH // NBUF - 1, quad, 0)

    for b in range(NBUF):
      pltpu.make_async_copy(t_in.at[sidx[b]], rows[b], sems[b]).wait()
      pltpu.sync_copy(rows[b], acc.at[didx[b]], add=True)

    plsc.subcore_barrier()
    pltpu.sync_copy(acc.at[pl.ds(zb, ZROWS)], t_out.at[pl.ds(hb, ZROWS)])

  return _layer


N_GATHER = BATCH * (1 + K_CAND)
GPT = N_GATHER // NW
GCH = GPT // 2


@functools.cache
def _make_gather4():
  mesh = plsc.VectorSubcoreMesh(
      core_axis_name="c", subcore_axis_name="s",
      num_cores=NC, num_subcores=NS)

  @functools.partial(
      pl.kernel,
      out_type=[jax.ShapeDtypeStruct((N_GATHER, DIM), jnp.float32)] * 4,
      mesh=mesh,
      scratch_types=[
          pltpu.VMEM((GCH,), jnp.int32),
          pltpu.VMEM((GCH,), jnp.int32),
          pltpu.VMEM((GCH, DIM), jnp.float32),
          pltpu.VMEM((GCH, DIM), jnp.float32),
          pltpu.SemaphoreType.DMA,
          pltpu.SemaphoreType.DMA,
      ],
  )
  def _gather4(t0, t1, t2, t3, gidx, g0, g1, g2, g3,
               idx_a, idx_b, rows_a, rows_b, sem_a, sem_b):
    wid = lax.axis_index("c") * NS + lax.axis_index("s")
    base_a = wid * GPT
    base_b = base_a + GCH
    pltpu.sync_copy(gidx.at[pl.ds(base_a, GCH)], idx_a)
    pltpu.sync_copy(gidx.at[pl.ds(base_b, GCH)], idx_b)
    tables = (t0, t1, t2, t3)
    outs = (g0, g1, g2, g3)
    steps = [(tables[j % 4], outs[j % 4],
              idx_a if j < 4 else idx_b,
              base_a if j < 4 else base_b) for j in range(8)]
    bufs = ((rows_a, sem_a), (rows_b, sem_b))
    t_hbm, _, idx, _ = steps[0]
    pltpu.async_copy(t_hbm.at[idx], rows_a, sem_a)
    for j in range(8):
      rows, sem = bufs[j % 2]
      if j < 7:
        t_n, _, idx_n, _ = steps[j + 1]
        rows_n, sem_n = bufs[(j + 1) % 2]
        pltpu.async_copy(t_n.at[idx_n], rows_n, sem_n)
      t_hbm, g_hbm, idx, base = steps[j]
      pltpu.make_async_copy(t_hbm.at[idx], rows, sem).wait()
      pltpu.sync_copy(rows, g_hbm.at[pl.ds(base, GCH)])

  return _gather4


def _finalize(g0, g1, g2, g3, label, tot_ref, scores_ref, rec_ref, emb_ref):
    u = 0.25 * (g0[0:BATCH, :] + g1[0:BATCH, :]
                + g2[0:BATCH, :] + g3[0:BATCH, :])
    reg = jnp.sum(u * u)
    cols = []
    for k in range(K_CAND):
        o = BATCH + k * BATCH
        ik = 0.25 * (g0[o:o + BATCH, :] + g1[o:o + BATCH, :]
                     + g2[o:o + BATCH, :] + g3[o:o + BATCH, :])
        reg = reg + jnp.sum(ik * ik)
        cols.append(jnp.sum(u * ik, axis=1, keepdims=True))
    scores = jnp.concatenate(cols, axis=1)

    m = jnp.max(scores, axis=1, keepdims=True)
    e = jnp.exp(scores - m)
    probs = e / jnp.sum(e, axis=1, keepdims=True)

    lbl = label[...]
    iota_k = lax.broadcasted_iota(jnp.int32, (BATCH, K_CAND), 1)
    lmax = jnp.max(lbl, axis=1, keepdims=True)
    tgt = jnp.min(jnp.where(lbl == lmax, iota_k, K_CAND),
                  axis=1, keepdims=True)

    m2 = jnp.max(probs, axis=1, keepdims=True)
    logp = (probs - m2
            - jnp.log(jnp.sum(jnp.exp(probs - m2), axis=1, keepdims=True)))
    chosen = jnp.sum(jnp.where(iota_k == tgt, logp, 0.0), axis=1)
    rec = -jnp.sum(chosen) / BATCH
    emb = L2_COEF * reg * 0.5 / BATCH

    scores_ref[...] = scores
    tot_ref[...] = jnp.reshape(rec + emb, (1, 1))
    rec_ref[...] = jnp.reshape(rec, (1, 1))
    emb_ref[...] = jnp.reshape(emb, (1, 1))


_finalize_call = pl.pallas_call(
    _finalize,
    out_shape=[
        jax.ShapeDtypeStruct((1, 1), jnp.float32),
        jax.ShapeDtypeStruct((BATCH, K_CAND), jnp.float32),
        jax.ShapeDtypeStruct((1, 1), jnp.float32),
        jax.ShapeDtypeStruct((1, 1), jnp.float32),
    ],
)


def kernel(user_index, candidate_news_index, label,
           user_emb, item_emb, edge_src, edge_dst):
    t0 = jnp.zeros((N_PAD, DIM), jnp.float32)
    t0 = lax.dynamic_update_slice(t0, user_emb, (0, 0))
    t0 = lax.dynamic_update_slice(t0, item_emb, (HALF, 0))

    esrc = edge_src.astype(jnp.int32)
    edst = edge_dst.astype(jnp.int32)
    esrc = esrc + jnp.where(esrc >= N_USERS, HALF - N_USERS, 0)
    edst = edst + jnp.where(edst >= N_USERS, HALF - N_USERS, 0)

    edst_loc = jnp.where(edst >= HALF, edst - HALF, edst)
    pad_src = (jnp.arange(NW * PADE, dtype=jnp.int32) % N_PAD).reshape(
        NW, PADE)
    pad_dst = (HALF + jnp.arange(NW * PADE, dtype=jnp.int32) % PADROWS
               ).reshape(NW, PADE)
    src_p = jnp.concatenate([esrc.reshape(NW, EPT), pad_src], axis=1)
    dst_p = jnp.concatenate([edst_loc.reshape(NW, EPT), pad_dst], axis=1)
    packed = src_p | (dst_p << 16)
    zin = jnp.zeros((ZROWS, DIM), jnp.float32)

    layer_fn = _make_layer()
    t1 = layer_fn(t0, packed, zin)
    t2 = layer_fn(t1, packed, zin)
    t3 = layer_fn(t2, packed, zin)

    gidx = jnp.concatenate(
        [user_index.astype(jnp.int32)]
        + [HALF + candidate_news_index[:, k].astype(jnp.int32)
           for k in range(K_CAND)])
    g0, g1, g2, g3 = _make_gather4()(t0, t1, t2, t3, gidx)

    tot, scores, rec, emb = _finalize_call(g0, g1, g2, g3, label)
    return (tot[0, 0], scores, rec[0, 0], emb[0, 0])
